# Initial kernel scaffold; baseline (speedup 1.0000x reference)
import jax
import jax.numpy as jnp
from jax.experimental import pallas as pl


def kernel(x, edge_index, Wl0, bl0, Wr0, br0, att0, bias0, Wl1, bl1, Wr1, br1, att1, bias1, Wl2, bl2, Wr2, br2, att2, bias2):
    raise NotImplementedError("write your pallas kernel here")


# speedup vs baseline: 11.8055x; 11.8055x over previous
import dataclasses
import functools

import jax
import jax.numpy as jnp
from jax import lax
from jax.experimental import pallas as pl
from jax.experimental.pallas import tpu as pltpu
from jax.experimental.pallas import tpu_sc as plsc

F32 = jnp.float32
I32 = jnp.int32

NN = 10000
NP = 10240
FD = 128
EE = 330000
NW = 32
CH = 128
NCHUNK = 82
PER_W = NCHUNK * CH
EP = NW * PER_W
DUMMY = NN
RPT = NP // 16

_mesh = plsc.VectorSubcoreMesh(core_axis_name="c", subcore_axis_name="s")

_sc_params = pltpu.CompilerParams()
if "needs_layout_passes" in pltpu.CompilerParams.__dataclass_fields__:
    _sc_params = dataclasses.replace(_sc_params, needs_layout_passes=False)


def _proj_body(h_ref, wl_ref, bl_ref, wr_ref, br_ref, xl_ref, xr_ref):
    h = h_ref[...]
    xl_ref[...] = jnp.dot(h, wl_ref[...], preferred_element_type=F32) + bl_ref[...]
    xr_ref[...] = jnp.dot(h, wr_ref[...], preferred_element_type=F32) + br_ref[...]


def tc_proj_first(h, Wl, bl, Wr, br):
    blk = 2048
    return pl.pallas_call(
        _proj_body,
        grid=(NP // blk,),
        in_specs=[
            pl.BlockSpec((blk, FD), lambda i: (i, 0)),
            pl.BlockSpec((FD, FD), lambda i: (0, 0)),
            pl.BlockSpec((1, FD), lambda i: (0, 0)),
            pl.BlockSpec((FD, FD), lambda i: (0, 0)),
            pl.BlockSpec((1, FD), lambda i: (0, 0)),
        ],
        out_specs=[pl.BlockSpec((blk, FD), lambda i: (i, 0))] * 2,
        out_shape=[jax.ShapeDtypeStruct((NP, FD), F32)] * 2,
    )(h, Wl, bl.reshape(1, FD), Wr, br.reshape(1, FD))


def _proj_next_body(p0_ref, p1_ref, b_ref, wl_ref, bl_ref, wr_ref, br_ref,
                    xl_ref, xr_ref):
    h = p0_ref[...] + p1_ref[...] + b_ref[...]
    h = jnp.where(h > 0, h, jnp.exp(jnp.minimum(h, 0.0)) - 1.0)
    xl_ref[...] = jnp.dot(h, wl_ref[...], preferred_element_type=F32) + bl_ref[...]
    xr_ref[...] = jnp.dot(h, wr_ref[...], preferred_element_type=F32) + br_ref[...]


def tc_proj_next(p0, p1, bias, Wl, bl, Wr, br):
    blk = 2048
    return pl.pallas_call(
        _proj_next_body,
        grid=(NP // blk,),
        in_specs=[
            pl.BlockSpec((blk, FD), lambda i: (i, 0)),
            pl.BlockSpec((blk, FD), lambda i: (i, 0)),
            pl.BlockSpec((1, FD), lambda i: (0, 0)),
            pl.BlockSpec((FD, FD), lambda i: (0, 0)),
            pl.BlockSpec((1, FD), lambda i: (0, 0)),
            pl.BlockSpec((FD, FD), lambda i: (0, 0)),
            pl.BlockSpec((1, FD), lambda i: (0, 0)),
        ],
        out_specs=[pl.BlockSpec((blk, FD), lambda i: (i, 0))] * 2,
        out_shape=[jax.ShapeDtypeStruct((NP, FD), F32)] * 2,
    )(p0, p1, bias.reshape(1, FD), Wl, bl.reshape(1, FD), Wr, br.reshape(1, FD))


def _alpha_body(nheads, gl_ref, gr_ref, attf_ref, a_ref, m_ref, mscr):
    i = pl.program_id(0)
    t = gl_ref[...] + gr_ref[...]
    t = jnp.where(t >= 0, t, 0.2 * t)
    ta = t * attf_ref[...]
    w = FD // nheads
    cols = [jnp.sum(ta[:, h * w:(h + 1) * w], axis=1, keepdims=True)
            for h in range(nheads)]
    a = cols[0] if nheads == 1 else jnp.concatenate(cols, axis=1)
    a_ref[...] = a
    bm = jnp.max(a)

    @pl.when(i == 0)
    def _():
        mscr[0] = bm

    @pl.when(i > 0)
    def _():
        mscr[0] = jnp.maximum(mscr[0], bm)

    m_ref[...] = jnp.broadcast_to(mscr[0], (1, 1))


def tc_alpha(gl, gr, attf, nheads):
    blk = 2048
    return pl.pallas_call(
        functools.partial(_alpha_body, nheads),
        grid=(EP // blk,),
        in_specs=[
            pl.BlockSpec((blk, FD), lambda i: (i, 0)),
            pl.BlockSpec((blk, FD), lambda i: (i, 0)),
            pl.BlockSpec((1, FD), lambda i: (0, 0)),
        ],
        out_specs=[
            pl.BlockSpec((blk, nheads), lambda i: (i, 0)),
            pl.BlockSpec((1, 1), lambda i: (0, 0)),
        ],
        out_shape=[
            jax.ShapeDtypeStruct((EP, nheads), F32),
            jax.ShapeDtypeStruct((1, 1), F32),
        ],
        scratch_shapes=[pltpu.SMEM((1,), F32)],
    )(gl, gr, attf)


def _exp_body(a_ref, m_ref, s_ref):
    s_ref[...] = jnp.exp(a_ref[...] - m_ref[0, 0])


def tc_exp(a, m, nheads):
    blk = 4096
    return pl.pallas_call(
        _exp_body,
        grid=(EP // blk,),
        in_specs=[
            pl.BlockSpec((blk, nheads), lambda i: (i, 0)),
            pl.BlockSpec((1, 1), lambda i: (0, 0)),
        ],
        out_specs=pl.BlockSpec((blk, nheads), lambda i: (i, 0)),
        out_shape=jax.ShapeDtypeStruct((EP, nheads), F32),
    )(a, m)


def _dsum_body(dp_ref, d_ref):
    d_ref[...] = jnp.sum(dp_ref[...], axis=0, keepdims=True)


def tc_dsum(dparts, nh_tot):
    blk = 5120
    return pl.pallas_call(
        _dsum_body,
        grid=(nh_tot // blk,),
        in_specs=[pl.BlockSpec((NW, blk), lambda i: (0, i))],
        out_specs=pl.BlockSpec((1, blk), lambda i: (0, i)),
        out_shape=jax.ShapeDtypeStruct((1, nh_tot), F32),
    )(dparts)


def _premul_body(nheads, gl_ref, c_ref, glw_ref):
    w = FD // nheads
    gl = gl_ref[...]
    c = c_ref[...]
    parts = [gl[:, h * w:(h + 1) * w] * c[:, h:h + 1] for h in range(nheads)]
    glw_ref[...] = parts[0] if nheads == 1 else jnp.concatenate(parts, axis=1)


def tc_premul(gl, coef, nheads):
    blk = 2048
    return pl.pallas_call(
        functools.partial(_premul_body, nheads),
        grid=(EP // blk,),
        in_specs=[
            pl.BlockSpec((blk, FD), lambda i: (i, 0)),
            pl.BlockSpec((blk, nheads), lambda i: (i, 0)),
        ],
        out_specs=pl.BlockSpec((blk, FD), lambda i: (i, 0)),
        out_shape=jax.ShapeDtypeStruct((EP, FD), F32),
    )(gl, coef)


def _final_body(p0_ref, p1_ref, b_ref, o_ref):
    o_ref[...] = p0_ref[...] + p1_ref[...] + b_ref[...]


def tc_final(p0, p1, bias):
    blk = 2048
    return pl.pallas_call(
        _final_body,
        grid=(NP // blk,),
        in_specs=[
            pl.BlockSpec((blk, FD), lambda i: (i, 0)),
            pl.BlockSpec((blk, FD), lambda i: (i, 0)),
            pl.BlockSpec((1, FD), lambda i: (0, 0)),
        ],
        out_specs=pl.BlockSpec((blk, FD), lambda i: (i, 0)),
        out_shape=jax.ShapeDtypeStruct((NP, FD), F32),
    )(p0, p1, bias.reshape(1, FD))


def sc_gather(xl, xr, sd2):
    @functools.partial(
        pl.kernel, mesh=_mesh, compiler_params=_sc_params,
        out_type=[jax.ShapeDtypeStruct((EP, FD), F32)] * 2,
        scratch_types=[
            pltpu.VMEM((2, CH), I32), pltpu.VMEM((2, CH), I32),
            pltpu.VMEM((CH, FD), F32), pltpu.VMEM((CH, FD), F32),
            pltpu.VMEM((CH, FD), F32), pltpu.VMEM((CH, FD), F32),
            pltpu.SemaphoreType.DMA, pltpu.SemaphoreType.DMA,
            pltpu.SemaphoreType.DMA, pltpu.SemaphoreType.DMA,
        ],
    )
    def k(xl_hbm, xr_hbm, sd_hbm, gl_hbm, gr_hbm,
          i0, i1, bl0, br0, bl1, br1, sl0, sr0, sl1, sr1):
        wid = lax.axis_index("s") * 2 + lax.axis_index("c")
        c0 = wid * NCHUNK

        def fire(c, ib, bl, br, sl, sr):
            pltpu.sync_copy(sd_hbm.at[c], ib)
            pltpu.async_copy(xl_hbm.at[ib.at[0]], bl, sl)
            pltpu.async_copy(xr_hbm.at[ib.at[1]], br, sr)

        def drain(ib, bl, br, sl, sr, c):
            pltpu.make_async_copy(xl_hbm.at[ib.at[0]], bl, sl).wait()
            pltpu.make_async_copy(xr_hbm.at[ib.at[1]], br, sr).wait()
            pltpu.sync_copy(bl, gl_hbm.at[pl.ds(c * CH, CH)])
            pltpu.sync_copy(br, gr_hbm.at[pl.ds(c * CH, CH)])

        fire(c0, i0, bl0, br0, sl0, sr0)

        @pl.loop(0, NCHUNK // 2)
        def _(jj):
            ca = c0 + jj * 2
            cb = ca + 1
            fire(cb, i1, bl1, br1, sl1, sr1)
            drain(i0, bl0, br0, sl0, sr0, ca)

            @pl.when(jj < NCHUNK // 2 - 1)
            def _():
                fire(ca + 2, i0, bl0, br0, sl0, sr0)

            drain(i1, bl1, br1, sl1, sr1, cb)

    return k(xl, xr, sd2)


def sc_scatter_d(sflat, dstp, nheads):
    nh_tot = NP * nheads
    chh = CH * nheads
    ng = chh // 16
    epg = 16 // nheads

    @functools.partial(
        pl.kernel, mesh=_mesh, compiler_params=_sc_params,
        out_type=jax.ShapeDtypeStruct((NW, nh_tot), F32),
        scratch_types=[
            pltpu.VMEM((nh_tot,), F32), pltpu.VMEM((CH,), I32),
            pltpu.VMEM((chh,), F32),
        ],
    )
    def k(s_hbm, dst_hbm, dp_hbm, dacc, di, sv):
        wid = lax.axis_index("s") * 2 + lax.axis_index("c")
        base = wid * PER_W

        @pl.loop(0, nh_tot, step=16)
        def _(i):
            dacc[pl.ds(i, 16)] = jnp.zeros((16,), F32)

        lane = lax.iota(I32, 16)
        eoff = lane // nheads
        hoff = lane - eoff * nheads

        @pl.loop(0, NCHUNK)
        def _(j):
            off = base + j * CH
            pltpu.sync_copy(dst_hbm.at[pl.ds(off, CH)], di)
            pltpu.sync_copy(s_hbm.at[pl.ds(off * nheads, chh)], sv)

            @pl.loop(0, ng)
            def _(g):
                el = plsc.load_gather(di, [g * epg + eoff])
                idx = el * nheads + hoff
                plsc.addupdate_scatter(dacc, [idx], sv[pl.ds(g * 16, 16)])

        pltpu.sync_copy(dacc, dp_hbm.at[wid])

    return k(sflat, dstp)


def sc_coef(sflat, dstp, d, nheads):
    nh_tot = NP * nheads
    chh = CH * nheads
    ng = chh // 16
    epg = 16 // nheads

    @functools.partial(
        pl.kernel, mesh=_mesh, compiler_params=_sc_params,
        out_type=jax.ShapeDtypeStruct((EP * nheads,), F32),
        scratch_types=[
            pltpu.VMEM((nh_tot,), F32), pltpu.VMEM((CH,), I32),
            pltpu.VMEM((chh,), F32), pltpu.VMEM((chh,), F32),
        ],
    )
    def k(s_hbm, dst_hbm, d_hbm, coef_hbm, dv, di, sv, cv):
        wid = lax.axis_index("s") * 2 + lax.axis_index("c")
        base = wid * PER_W
        pltpu.sync_copy(d_hbm.at[0], dv)

        lane = lax.iota(I32, 16)
        eoff = lane // nheads
        hoff = lane - eoff * nheads

        @pl.loop(0, NCHUNK)
        def _(j):
            off = base + j * CH
            pltpu.sync_copy(dst_hbm.at[pl.ds(off, CH)], di)
            pltpu.sync_copy(s_hbm.at[pl.ds(off * nheads, chh)], sv)

            @pl.loop(0, ng)
            def _(g):
                el = plsc.load_gather(di, [g * epg + eoff])
                idx = el * nheads + hoff
                dd = plsc.load_gather(dv, [idx])
                cv[pl.ds(g * 16, 16)] = sv[pl.ds(g * 16, 16)] / (dd + 1e-16)

            pltpu.sync_copy(cv, coef_hbm.at[pl.ds(off * nheads, chh)])

    return k(sflat, dstp, d)


def sc_out_scatter(glw, sd2, znp):
    @functools.partial(
        pl.kernel, mesh=_mesh, compiler_params=_sc_params,
        out_type=jax.ShapeDtypeStruct((2, NP, FD), F32),
        scratch_types=[
            pltpu.VMEM_SHARED((NP, FD), F32),
            pltpu.VMEM((2, CH), I32), pltpu.VMEM((2, CH), I32),
            pltpu.VMEM((CH, FD), F32), pltpu.VMEM((CH, FD), F32),
            pltpu.SemaphoreType.DMA, pltpu.SemaphoreType.DMA,
        ],
    )
    def k(glw_hbm, sd_hbm, z_hbm, op_hbm, acc_sh, i0, i1, b0, b1, s0, s1):
        cid = lax.axis_index("c")
        sid = lax.axis_index("s")
        wid = sid * 2 + cid
        c0 = wid * NCHUNK
        r0 = sid * RPT
        pltpu.sync_copy(z_hbm.at[pl.ds(r0, RPT)], acc_sh.at[pl.ds(r0, RPT)])

        def fire(c, ib, buf, sem):
            pltpu.sync_copy(sd_hbm.at[c], ib)
            pltpu.async_copy(glw_hbm.at[pl.ds(c * CH, CH)], buf, sem)

        def drain(ib, buf, sem):
            pltpu.make_async_copy(glw_hbm.at[pl.ds(0, CH)], buf, sem).wait()
            pltpu.sync_copy(buf, acc_sh.at[ib.at[1]], add=True)

        plsc.subcore_barrier()
        fire(c0, i0, b0, s0)

        @pl.loop(0, NCHUNK // 2)
        def _(jj):
            ca = c0 + jj * 2
            cb = ca + 1
            fire(cb, i1, b1, s1)
            drain(i0, b0, s0)

            @pl.when(jj < NCHUNK // 2 - 1)
            def _():
                fire(ca + 2, i0, b0, s0)

            drain(i1, b1, s1)

        plsc.subcore_barrier()
        pltpu.sync_copy(acc_sh.at[pl.ds(r0, RPT)],
                        op_hbm.at[cid].at[pl.ds(r0, RPT)])

    return k(glw, sd2, znp)


def kernel(x, edge_index, Wl0, bl0, Wr0, br0, att0, bias0,
           Wl1, bl1, Wr1, br1, att1, bias1,
           Wl2, bl2, Wr2, br2, att2, bias2):
    loop = jnp.arange(NN, dtype=I32)
    padi = jnp.full((EP - EE,), DUMMY, I32)
    srcp = jnp.concatenate([edge_index[0], loop, padi])
    dstp = jnp.concatenate([edge_index[1], loop, padi])
    sd2 = jnp.stack([srcp.reshape(EP // CH, CH),
                     dstp.reshape(EP // CH, CH)], axis=1)
    h = jnp.pad(x, ((0, NP - NN), (0, 0)))
    znp = jnp.zeros((NP, FD), F32)

    layers = [
        (Wl0, bl0, Wr0, br0, att0, bias0, 4),
        (Wl1, bl1, Wr1, br1, att1, bias1, 4),
        (Wl2, bl2, Wr2, br2, att2, bias2, 1),
    ]
    p0 = p1 = None
    prev_bias = None
    for li, (Wl, bl, Wr, br, att, bias, nheads) in enumerate(layers):
        if li == 0:
            xl, xr = tc_proj_first(h, Wl, bl, Wr, br)
        else:
            xl, xr = tc_proj_next(p0, p1, prev_bias, Wl, bl, Wr, br)
        gl, gr = sc_gather(xl, xr, sd2)
        a, m = tc_alpha(gl, gr, att.reshape(1, FD), nheads)
        s = tc_exp(a, m, nheads)
        sflat = s.reshape(-1)
        dparts = sc_scatter_d(sflat, dstp, nheads)
        d = tc_dsum(dparts, NP * nheads)
        coef = sc_coef(sflat, dstp, d, nheads).reshape(EP, nheads)
        glw = tc_premul(gl, coef, nheads)
        pp = sc_out_scatter(glw, sd2, znp)
        p0, p1 = pp[0], pp[1]
        prev_bias = bias

    out = tc_final(p0, p1, bias2)
    return out[:NN]
